# Initial kernel scaffold; baseline (speedup 1.0000x reference)
import jax
import jax.numpy as jnp
from jax.experimental import pallas as pl


def kernel(x, table):
    raise NotImplementedError("write your pallas kernel here")


# speedup vs baseline: 3.4817x; 3.4817x over previous
import functools

import jax
import jax.numpy as jnp
from jax import lax
from jax.experimental import pallas as pl
from jax.experimental.pallas import tpu as pltpu
from jax.experimental.pallas import tpu_sc as plsc

NC = 2
NS = 16
NW = NC * NS
CHUNK = 512


def _gather_sc(table, idx_flat):
    n = idx_flat.shape[0]
    embed = table.shape[1]
    per_w = n // NW
    chunks = per_w // CHUNK
    mesh = plsc.VectorSubcoreMesh(core_axis_name="c", subcore_axis_name="s")

    table_pad = jnp.pad(table, ((0, 0), (0, 128 - embed)))

    @functools.partial(
        pl.kernel,
        mesh=mesh,
        out_type=jax.ShapeDtypeStruct((n, 128), jnp.float32),
        scratch_types=[
            pltpu.VMEM((CHUNK,), jnp.int32),
            pltpu.VMEM((CHUNK, 128), jnp.float32),
            pltpu.SemaphoreType.DMA,
        ],
    )
    def k(table_hbm, idx_hbm, out_hbm, idx_v, rows_v, sem):
        wid = lax.axis_index("s") * NC + lax.axis_index("c")
        base = wid * per_w

        @pl.loop(0, chunks)
        def _(c):
            off = base + c * CHUNK
            pltpu.sync_copy(idx_hbm.at[pl.ds(off, CHUNK)], idx_v)
            pltpu.async_copy(table_hbm.at[idx_v], rows_v, sem).wait()
            pltpu.sync_copy(rows_v, out_hbm.at[pl.ds(off, CHUNK)])

    return k(table_pad, idx_flat)


def _transpose_tc(emb, e):
    b, l, ep = emb.shape
    bb = 32

    def body(x_ref, o_ref):
        for i in range(bb):
            o_ref[i] = x_ref[i, :, 0:e].T

    return pl.pallas_call(
        body,
        grid=(b // bb,),
        in_specs=[pl.BlockSpec((bb, l, ep), lambda i: (i, 0, 0))],
        out_specs=pl.BlockSpec((bb, e, l), lambda i: (i, 0, 0)),
        out_shape=jax.ShapeDtypeStruct((b, e, l), jnp.float32),
    )(emb)


def kernel(x, table):
    b, l = x.shape
    e = table.shape[1]
    idx = x.reshape(-1).astype(jnp.int32)
    emb = _gather_sc(table, idx)
    return _transpose_tc(emb.reshape(b, l, 128), e)


# speedup vs baseline: 3.6523x; 1.0490x over previous
import functools

import jax
import jax.numpy as jnp
from jax import lax
from jax.experimental import pallas as pl
from jax.experimental.pallas import tpu as pltpu
from jax.experimental.pallas import tpu_sc as plsc

NC = 2
NS = 16
NW = NC * NS
CHUNK = 256


def _gather_sc(table, idx_flat):
    n = idx_flat.shape[0]
    embed = table.shape[1]
    per_w = n // NW
    chunks = per_w // CHUNK
    mesh = plsc.VectorSubcoreMesh(core_axis_name="c", subcore_axis_name="s")

    table_pad = jnp.pad(table, ((0, 0), (0, 128 - embed)))
    idx2 = idx_flat.reshape(1, n)

    @functools.partial(
        pl.kernel,
        mesh=mesh,
        out_type=jax.ShapeDtypeStruct((n, 128), jnp.float32),
    )
    def k(table_hbm, idx_hbm, out_hbm):
        def body(i_vmem, o_vmem):
            pltpu.sync_copy(table_hbm.at[i_vmem.at[0]], o_vmem)

        pltpu.emit_pipeline(
            body,
            grid=(n // CHUNK,),
            in_specs=[pl.BlockSpec((1, CHUNK), index_map=lambda i: (0, i))],
            out_specs=[pl.BlockSpec((CHUNK, 128), index_map=lambda i: (i, 0))],
            core_axis_name=("c", "s"),
            dimension_semantics=(pltpu.PARALLEL,),
        )(idx_hbm, out_hbm)

    return k(table_pad, idx2)


def _transpose_tc(emb, e):
    b, l, ep = emb.shape
    bb = 32

    def body(x_ref, o_ref):
        for i in range(bb):
            o_ref[i] = x_ref[i, :, 0:e].T

    return pl.pallas_call(
        body,
        grid=(b // bb,),
        in_specs=[pl.BlockSpec((bb, l, ep), lambda i: (i, 0, 0))],
        out_specs=pl.BlockSpec((bb, e, l), lambda i: (i, 0, 0)),
        out_shape=jax.ShapeDtypeStruct((b, e, l), jnp.float32),
    )(emb)


def kernel(x, table):
    b, l = x.shape
    e = table.shape[1]
    idx = x.reshape(-1).astype(jnp.int32)
    emb = _gather_sc(table, idx)
    return _transpose_tc(emb.reshape(b, l, 128), e)
